# baseline (speedup 1.0000x reference)
import functools

import jax
import jax.numpy as jnp
from jax import lax
from jax.experimental import pallas as pl
from jax.experimental.pallas import tpu as pltpu
from jax.experimental.pallas import tpu_sc as plsc

VOCAB = 100000
EMBED = 64
BATCH = 1024
CTX = 50

NC, NS, L = 2, 16, 16
NW = NC * NS
ROWS_PER_W = BATCH // NW
IDX_PER_W = ROWS_PER_W * CTX
GCHUNK = 80
NCHUNK = IDX_PER_W // GCHUNK

VBLK = 2048
NV = (VOCAB + VBLK - 1) // VBLK


def _sc_hidden_body(table_hbm, idx_hbm, hid_hbm, idx_v, rows_v, hid_v, sem):
    wid = lax.axis_index("s") * NC + lax.axis_index("c")
    pltpu.sync_copy(idx_hbm.at[wid], idx_v)
    handles = []
    for c in range(NCHUNK):
        handles.append(
            pltpu.async_copy(
                table_hbm.at[idx_v.at[c]],
                rows_v.at[pl.ds(c * GCHUNK, GCHUNK)],
                sem,
            )
        )
    for h in handles:
        h.wait()

    inv = jnp.float32(1.0 / CTX)

    def row_body(r, carry):
        def j_body(j, accs):
            b = r * CTX + j
            return tuple(
                accs[c] + rows_v[b, pl.ds(c * L, L)] for c in range(EMBED // L)
            )

        zeros = tuple(
            jnp.zeros((L,), jnp.float32) for _ in range(EMBED // L)
        )
        accs = lax.fori_loop(0, CTX, j_body, zeros)
        for c in range(EMBED // L):
            hid_v[r, pl.ds(c * L, L)] = accs[c] * inv
        return carry

    lax.fori_loop(0, ROWS_PER_W, row_body, 0)
    pltpu.sync_copy(hid_v, hid_hbm.at[pl.ds(wid * ROWS_PER_W, ROWS_PER_W)])


@functools.cache
def _sc_hidden():
    return pl.kernel(
        _sc_hidden_body,
        out_type=jax.ShapeDtypeStruct((BATCH, EMBED), jnp.float32),
        mesh=plsc.VectorSubcoreMesh(
            core_axis_name="c",
            subcore_axis_name="s",
            num_cores=NC,
            num_subcores=NS,
        ),
        scratch_types=[
            pltpu.VMEM((NCHUNK, GCHUNK), jnp.int32),
            pltpu.VMEM((IDX_PER_W, EMBED), jnp.float32),
            pltpu.VMEM((ROWS_PER_W, EMBED), jnp.float32),
            pltpu.SemaphoreType.DMA,
        ],
        compiler_params=pltpu.CompilerParams(use_tc_tiling_on_sc=False),
    )


def _fused_body(hid_ref, w_ref, b_ref, raw_ref, m_ref, s_ref):
    vb = pl.program_id(0)

    @pl.when(vb == 0)
    def _():
        m_ref[...] = jnp.full((BATCH, 1), -1e30, jnp.float32)
        s_ref[...] = jnp.zeros((BATCH, 1), jnp.float32)

    h = hid_ref[...].astype(jnp.bfloat16)
    w = w_ref[...].astype(jnp.bfloat16)
    logits = lax.dot_general(
        h, w, (((1,), (1,)), ((), ())), preferred_element_type=jnp.float32
    )
    logits = logits + b_ref[...]
    u = jax.lax.bitcast_convert_type(logits, jnp.int32) + 0x8000
    u_lo = jax.lax.shift_right_logical(u[:, : VBLK // 2], 16)
    u_hi = u[:, VBLK // 2 :] & jnp.int32(-65536)
    raw_ref[...] = u_lo | u_hi
    col = lax.broadcasted_iota(jnp.int32, (1, VBLK), 1) + vb * VBLK
    logits = jnp.where(col < VOCAB, logits, -jnp.inf)
    tmax = jnp.max(logits, axis=1, keepdims=True)
    tsum = jnp.sum(jnp.exp(logits - tmax), axis=1, keepdims=True)
    m_old = m_ref[...]
    m_new = jnp.maximum(m_old, tmax)
    s_ref[...] = s_ref[...] * jnp.exp(m_old - m_new) + tsum * jnp.exp(
        tmax - m_new
    )
    m_ref[...] = m_new


def kernel(inputs, emb_table, out_W, out_b):
    idx = inputs.reshape(NW, NCHUNK, GCHUNK)
    hidden = _sc_hidden()(emb_table, idx)

    hid_spec = pl.BlockSpec((BATCH, EMBED), lambda v: (0, 0))
    w_spec = pl.BlockSpec((VBLK, EMBED), lambda v: (v, 0))
    b_spec = pl.BlockSpec((1, VBLK), lambda v: (0, v))
    ms_spec = pl.BlockSpec((BATCH, 1), lambda v: (0, 0))

    raw, m, sum_exp = pl.pallas_call(
        _fused_body,
        grid=(NV,),
        in_specs=[hid_spec, w_spec, b_spec],
        out_specs=[
            pl.BlockSpec((BATCH, VBLK // 2), lambda v: (0, v)),
            ms_spec,
            ms_spec,
        ],
        out_shape=[
            jax.ShapeDtypeStruct((BATCH, NV * VBLK // 2), jnp.int32),
            jax.ShapeDtypeStruct((BATCH, 1), jnp.float32),
            jax.ShapeDtypeStruct((BATCH, 1), jnp.float32),
        ],
        compiler_params=pltpu.CompilerParams(
            dimension_semantics=("arbitrary",)
        ),
    )(hidden, out_W, out_b.reshape(1, VOCAB))

    return _finalize(raw, m, sum_exp)


def _finalize(raw_packed, m, sum_exp):
    lo = jax.lax.bitcast_convert_type(
        jax.lax.shift_left(raw_packed, 16), jnp.float32
    ).reshape(BATCH, NV, VBLK // 2)
    hi = jax.lax.bitcast_convert_type(
        raw_packed & jnp.int32(-65536), jnp.float32
    ).reshape(BATCH, NV, VBLK // 2)
    full = jnp.concatenate([lo, hi], axis=2).reshape(BATCH, NV * VBLK)
    lse = m + jnp.log(sum_exp)
    return full[:, :VOCAB] - lse


# speedup vs baseline: 1.8110x; 1.8110x over previous
import functools

import jax
import jax.numpy as jnp
from jax import lax
from jax.experimental import pallas as pl
from jax.experimental.pallas import tpu as pltpu
from jax.experimental.pallas import tpu_sc as plsc

VOCAB = 100000
EMBED = 64
BATCH = 1024
CTX = 50

NC, NS, L = 2, 16, 16
NW = NC * NS
ROWS_PER_W = BATCH // NW
IDX_PER_W = ROWS_PER_W * CTX
GCHUNK = 80
NCHUNK = IDX_PER_W // GCHUNK

VBLK = 2048
NV = (VOCAB + VBLK - 1) // VBLK


def _sc_hidden_body(table_hbm, idx_hbm, hid_hbm, idx_v, rows_v, hid_v, sem):
    wid = lax.axis_index("s") * NC + lax.axis_index("c")
    pltpu.sync_copy(idx_hbm.at[wid], idx_v)
    handles = []
    for c in range(NCHUNK):
        handles.append(
            pltpu.async_copy(
                table_hbm.at[idx_v.at[c]],
                rows_v.at[pl.ds(c * GCHUNK, GCHUNK)],
                sem,
            )
        )
    for h in handles:
        h.wait()

    inv = jnp.float32(1.0 / CTX)

    def row_body(r, carry):
        def j_body(j, accs):
            b = r * CTX + j
            return tuple(
                accs[c] + rows_v[b, pl.ds(c * L, L)] for c in range(EMBED // L)
            )

        zeros = tuple(
            jnp.zeros((L,), jnp.float32) for _ in range(EMBED // L)
        )
        accs = lax.fori_loop(0, CTX, j_body, zeros)
        for c in range(EMBED // L):
            hid_v[r, pl.ds(c * L, L)] = accs[c] * inv
        return carry

    lax.fori_loop(0, ROWS_PER_W, row_body, 0)
    pltpu.sync_copy(hid_v, hid_hbm.at[pl.ds(wid * ROWS_PER_W, ROWS_PER_W)])


@functools.cache
def _sc_hidden():
    return pl.kernel(
        _sc_hidden_body,
        out_type=jax.ShapeDtypeStruct((BATCH, EMBED), jnp.float32),
        mesh=plsc.VectorSubcoreMesh(
            core_axis_name="c",
            subcore_axis_name="s",
            num_cores=NC,
            num_subcores=NS,
        ),
        scratch_types=[
            pltpu.VMEM((NCHUNK, GCHUNK), jnp.int32),
            pltpu.VMEM((IDX_PER_W, EMBED), jnp.float32),
            pltpu.VMEM((ROWS_PER_W, EMBED), jnp.float32),
            pltpu.SemaphoreType.DMA,
        ],
        compiler_params=pltpu.CompilerParams(use_tc_tiling_on_sc=False),
    )


def _fused_body(hid_ref, w_ref, b_ref, raw_ref, m_ref, s_ref):
    vb = pl.program_id(0)

    @pl.when(vb == 0)
    def _():
        m_ref[...] = jnp.full((BATCH, 1), -1e30, jnp.float32)
        s_ref[...] = jnp.zeros((BATCH, 1), jnp.float32)

    h = hid_ref[...].astype(jnp.bfloat16)
    w = w_ref[...].astype(jnp.bfloat16)
    logits = lax.dot_general(
        h, w, (((1,), (1,)), ((), ())), preferred_element_type=jnp.float32
    )
    logits = logits + b_ref[...]
    raw_ref[...] = logits.astype(jnp.bfloat16)
    col = lax.broadcasted_iota(jnp.int32, (1, VBLK), 1) + vb * VBLK
    logits = jnp.where(col < VOCAB, logits, -jnp.inf)
    tmax = jnp.max(logits, axis=1, keepdims=True)
    tsum = jnp.sum(jnp.exp(logits - tmax), axis=1, keepdims=True)
    m_old = m_ref[...]
    m_new = jnp.maximum(m_old, tmax)
    s_ref[...] = s_ref[...] * jnp.exp(m_old - m_new) + tsum * jnp.exp(
        tmax - m_new
    )
    m_ref[...] = m_new


def kernel(inputs, emb_table, out_W, out_b):
    idx = inputs.reshape(NW, NCHUNK, GCHUNK)
    hidden = _sc_hidden()(emb_table, idx)

    hid_spec = pl.BlockSpec((BATCH, EMBED), lambda v: (0, 0))
    w_spec = pl.BlockSpec((VBLK, EMBED), lambda v: (v, 0))
    b_spec = pl.BlockSpec((1, VBLK), lambda v: (0, v))
    ms_spec = pl.BlockSpec((BATCH, 1), lambda v: (0, 0))

    raw, m, sum_exp = pl.pallas_call(
        _fused_body,
        grid=(NV,),
        in_specs=[hid_spec, w_spec, b_spec],
        out_specs=[
            pl.BlockSpec((BATCH, VBLK), lambda v: (0, v)),
            ms_spec,
            ms_spec,
        ],
        out_shape=[
            jax.ShapeDtypeStruct((BATCH, NV * VBLK), jnp.bfloat16),
            jax.ShapeDtypeStruct((BATCH, 1), jnp.float32),
            jax.ShapeDtypeStruct((BATCH, 1), jnp.float32),
        ],
        compiler_params=pltpu.CompilerParams(
            dimension_semantics=("arbitrary",)
        ),
    )(hidden, out_W, out_b.reshape(1, VOCAB))

    return _finalize(raw, m, sum_exp)


def _finalize(raw_pad, m, sum_exp):
    lse = m + jnp.log(sum_exp)
    return raw_pad[:, :VOCAB].astype(jnp.float32) - lse


# speedup vs baseline: 2.2277x; 1.2301x over previous
import functools

import jax
import jax.numpy as jnp
from jax import lax
from jax.experimental import pallas as pl
from jax.experimental.pallas import tpu as pltpu
from jax.experimental.pallas import tpu_sc as plsc

VOCAB = 100000
EMBED = 64
BATCH = 1024
CTX = 50

NC, NS, L = 2, 16, 16
NW = NC * NS
ROWS_PER_W = BATCH // NW
IDX_PER_W = ROWS_PER_W * CTX
GCHUNK = 80
NCHUNK = IDX_PER_W // GCHUNK

VBLK = 2048
NV = (VOCAB + VBLK - 1) // VBLK


def _sc_hidden_body(table_hbm, idx_hbm, hid_hbm, idx_v, rows_v, hid_v, sem):
    wid = lax.axis_index("s") * NC + lax.axis_index("c")
    pltpu.sync_copy(idx_hbm.at[wid], idx_v)
    handles = []
    for c in range(NCHUNK):
        handles.append(
            pltpu.async_copy(
                table_hbm.at[idx_v.at[c]],
                rows_v.at[pl.ds(c * GCHUNK, GCHUNK)],
                sem,
            )
        )
    for h in handles:
        h.wait()

    inv = jnp.float32(1.0 / CTX)

    def row_body(r, carry):
        def j_body(j, accs):
            b = r * CTX + j
            return tuple(
                accs[c] + rows_v[b, pl.ds(c * L, L)] for c in range(EMBED // L)
            )

        zeros = tuple(
            jnp.zeros((L,), jnp.float32) for _ in range(EMBED // L)
        )
        accs = lax.fori_loop(0, CTX, j_body, zeros)
        for c in range(EMBED // L):
            hid_v[r, pl.ds(c * L, L)] = accs[c] * inv
        return carry

    lax.fori_loop(0, ROWS_PER_W, row_body, 0)
    pltpu.sync_copy(hid_v, hid_hbm.at[pl.ds(wid * ROWS_PER_W, ROWS_PER_W)])


@functools.cache
def _sc_hidden():
    return pl.kernel(
        _sc_hidden_body,
        out_type=jax.ShapeDtypeStruct((BATCH, EMBED), jnp.float32),
        mesh=plsc.VectorSubcoreMesh(
            core_axis_name="c",
            subcore_axis_name="s",
            num_cores=NC,
            num_subcores=NS,
        ),
        scratch_types=[
            pltpu.VMEM((NCHUNK, GCHUNK), jnp.int32),
            pltpu.VMEM((IDX_PER_W, EMBED), jnp.float32),
            pltpu.VMEM((ROWS_PER_W, EMBED), jnp.float32),
            pltpu.SemaphoreType.DMA,
        ],
        compiler_params=pltpu.CompilerParams(use_tc_tiling_on_sc=False),
    )


def _fused_body(hid_ref, w_ref, b_ref, raw_ref, m_ref, s_ref):
    vb = pl.program_id(0)

    @pl.when(vb == 0)
    def _():
        m_ref[...] = jnp.full((BATCH, 1), -1e30, jnp.float32)
        s_ref[...] = jnp.zeros((BATCH, 1), jnp.float32)

    h = hid_ref[...].astype(jnp.bfloat16)
    w = w_ref[...].astype(jnp.bfloat16)
    logits = lax.dot_general(
        h, w, (((1,), (1,)), ((), ())), preferred_element_type=jnp.float32
    )
    logits = logits + b_ref[...]
    raw_ref[...] = logits.astype(jnp.bfloat16)[None]
    col = lax.broadcasted_iota(jnp.int32, (1, VBLK), 1) + vb * VBLK
    logits = jnp.where(col < VOCAB, logits, -jnp.inf)
    tmax = jnp.max(logits, axis=1, keepdims=True)
    tsum = jnp.sum(jnp.exp(logits - tmax), axis=1, keepdims=True)
    m_old = m_ref[...]
    m_new = jnp.maximum(m_old, tmax)
    s_ref[...] = s_ref[...] * jnp.exp(m_old - m_new) + tsum * jnp.exp(
        tmax - m_new
    )
    m_ref[...] = m_new


def kernel(inputs, emb_table, out_W, out_b):
    idx = inputs.reshape(NW, NCHUNK, GCHUNK)
    hidden = _sc_hidden()(emb_table, idx)

    hid_spec = pl.BlockSpec((BATCH, EMBED), lambda v: (0, 0))
    w_spec = pl.BlockSpec((VBLK, EMBED), lambda v: (v, 0))
    b_spec = pl.BlockSpec((1, VBLK), lambda v: (0, v))
    ms_spec = pl.BlockSpec((BATCH, 1), lambda v: (0, 0))

    raw, m, sum_exp = pl.pallas_call(
        _fused_body,
        grid=(NV,),
        in_specs=[hid_spec, w_spec, b_spec],
        out_specs=[
            pl.BlockSpec((1, BATCH, VBLK), lambda v: (v, 0, 0)),
            ms_spec,
            ms_spec,
        ],
        out_shape=[
            jax.ShapeDtypeStruct((NV, BATCH, VBLK), jnp.bfloat16),
            jax.ShapeDtypeStruct((BATCH, 1), jnp.float32),
            jax.ShapeDtypeStruct((BATCH, 1), jnp.float32),
        ],
        compiler_params=pltpu.CompilerParams(
            dimension_semantics=("arbitrary",)
        ),
    )(hidden, out_W, out_b.reshape(1, VOCAB))

    return _finalize(raw, m, sum_exp)


def _finalize(raw3d, m, sum_exp):
    lse = m + jnp.log(sum_exp)
    full = raw3d.transpose(1, 0, 2).reshape(BATCH, NV * VBLK)
    return full[:, :VOCAB].astype(jnp.float32) - lse
